# baseline (speedup 1.0000x reference)
import jax
import jax.numpy as jnp
from jax.experimental import pallas as pl

_CHUNK = 64


def _cumsum_kernel(x_ref, tri_ref, o_ref):
    tri = tri_ref[...]
    nchunks = x_ref.shape[1] // _CHUNK
    carry = jnp.zeros((x_ref.shape[0], 1), dtype=jnp.float32)
    for k in range(nchunks):
        sl = pl.ds(k * _CHUNK, _CHUNK)
        chunk = x_ref[:, sl]
        within = jax.lax.dot(chunk, tri, preferred_element_type=jnp.float32)
        out = within + carry
        o_ref[:, sl] = out
        carry = out[:, _CHUNK - 1:_CHUNK]


def kernel(x):
    n, d = x.shape
    rb = 2048
    tri = jnp.triu(jnp.ones((_CHUNK, _CHUNK), dtype=jnp.float32))
    return pl.pallas_call(
        _cumsum_kernel,
        grid=(n // rb,),
        in_specs=[
            pl.BlockSpec((rb, d), lambda i: (i, 0)),
            pl.BlockSpec((_CHUNK, _CHUNK), lambda i: (0, 0)),
        ],
        out_specs=pl.BlockSpec((rb, d), lambda i: (i, 0)),
        out_shape=jax.ShapeDtypeStruct((n, d), jnp.float32),
    )(x, tri)


# speedup vs baseline: 1.7310x; 1.7310x over previous
import jax
import jax.numpy as jnp
from jax.experimental import pallas as pl

_CHUNK = 256


def _cumsum_kernel(x_ref, tri_ref, o_ref):
    tri = tri_ref[...]
    nchunks = x_ref.shape[1] // _CHUNK
    carry = jnp.zeros((x_ref.shape[0], 1), dtype=jnp.float32)
    for k in range(nchunks):
        sl = pl.ds(k * _CHUNK, _CHUNK)
        chunk = x_ref[:, sl]
        within = jax.lax.dot(chunk, tri, preferred_element_type=jnp.float32)
        out = within + carry
        o_ref[:, sl] = out
        carry = out[:, _CHUNK - 1:_CHUNK]


def kernel(x):
    n, d = x.shape
    rb = 2048
    tri = jnp.triu(jnp.ones((_CHUNK, _CHUNK), dtype=jnp.float32))
    return pl.pallas_call(
        _cumsum_kernel,
        grid=(n // rb,),
        in_specs=[
            pl.BlockSpec((rb, d), lambda i: (i, 0)),
            pl.BlockSpec((_CHUNK, _CHUNK), lambda i: (0, 0)),
        ],
        out_specs=pl.BlockSpec((rb, d), lambda i: (i, 0)),
        out_shape=jax.ShapeDtypeStruct((n, d), jnp.float32),
    )(x, tri)
